# baseline (speedup 1.0000x reference)
import functools

import jax
import jax.numpy as jnp
import numpy as np
from jax import lax
from jax.experimental import pallas as pl
from jax.experimental.pallas import tpu as pltpu
from jax.experimental.pallas import tpu_sc as plsc

_N_PTS = 262144
_IN_DIM = 3
_OUT_DIM = 3
_N_LEVELS = 16
_F = 2
_T = 2 ** 19
_MASK = _T - 1
_BASE_RES = 16.0
_AABB_MIN = -0.5
_PLS = float(np.exp2(np.log2(2048.0 * 0.5 / 16.0) / (16.0 - 1.0)))
_RES = [float(np.floor(_BASE_RES * _PLS ** l)) for l in range(_N_LEVELS)]
_P2 = int(np.uint32(2654435761).view(np.int32))
_P3 = int(np.uint32(805459861).view(np.int32))

_NC, _NS = 2, 16
_NW = _NC * _NS
_PW = _N_PTS // _NW
_C = 1024
_NCHUNK = _PW // _C
_G16 = _C // 16
_ENC_DIM = _N_LEVELS * _F


def _enc_body(x_h, y_h, z_h, tab_h, enc_h, xv, yv, zv, idx_v, wdup_v,
              rows_v, enc_v, sem):
    cid = lax.axis_index("c")
    sid = lax.axis_index("s")
    wid = sid * _NC + cid
    base0 = wid * _PW
    iota = lax.iota(jnp.int32, 16)
    pair = iota // 2
    feat = iota & 1

    def chunk(ci, carry):
        base = base0 + ci * _C
        pltpu.sync_copy(x_h.at[pl.ds(base, _C)], xv)
        pltpu.sync_copy(y_h.at[pl.ds(base, _C)], yv)
        pltpu.sync_copy(z_h.at[pl.ds(base, _C)], zv)
        for l in range(_N_LEVELS):
            res = _RES[l]
            lofs = l * _T

            def phase_a(g, c2, res=res, lofs=lofs):
                o = g * 16
                xs = xv[pl.ds(o, 16)]
                ys = yv[pl.ds(o, 16)]
                zs = zv[pl.ds(o, 16)]
                sx = (xs - _AABB_MIN) * res
                sy = (ys - _AABB_MIN) * res
                sz = (zs - _AABB_MIN) * res
                px = sx.astype(jnp.int32)
                py = sy.astype(jnp.int32)
                pz = sz.astype(jnp.int32)
                fx = sx - px.astype(jnp.float32)
                fy = sy - py.astype(jnp.float32)
                fz = sz - pz.astype(jnp.float32)
                hx = (px, px + 1)
                hy0 = py * _P2
                hz0 = pz * _P3
                hyz = ((hy0 ^ hz0, (hy0 + _P2) ^ hz0),
                       (hy0 ^ (hz0 + _P3), (hy0 + _P2) ^ (hz0 + _P3)))
                gx = (1.0 - fx, fx)
                gy = (1.0 - fy, fy)
                gz = (1.0 - fz, fz)
                wxy = [[gx[a] * gy[b] for b in range(2)] for a in range(2)]
                for c in range(8):
                    cx, cy, cz = c & 1, (c >> 1) & 1, (c >> 2) & 1
                    e = 2 * (((hx[cx] ^ hyz[cz][cy]) & _MASK) + lofs)
                    wb = (c * _C + o) * 2
                    plsc.store_scatter(idx_v, [wb + 2 * iota], e)
                    plsc.store_scatter(idx_v, [wb + 2 * iota + 1], e + 1)
                    w = wxy[cx][cy] * gz[cz]
                    plsc.store_scatter(wdup_v, [wb + 2 * iota], w)
                    plsc.store_scatter(wdup_v, [wb + 2 * iota + 1], w)
                return c2

            lax.fori_loop(0, _G16, phase_a, 0)
            pltpu.async_copy(tab_h.at[idx_v], rows_v, sem).wait()

            def phase_b(g, c2, l=l):
                o8 = g * 8
                acc = jnp.zeros((16,), jnp.float32)
                for c in range(8):
                    b = (c * _C + o8) * 2
                    acc = acc + wdup_v[pl.ds(b, 16)] * rows_v[pl.ds(b, 16)]
                plsc.store_scatter(enc_v, [o8 + pair, 2 * l + feat], acc)
                return c2

            lax.fori_loop(0, _C // 8, phase_b, 0)
        pltpu.sync_copy(enc_v, enc_h.at[pl.ds(base, _C)])
        return carry

    lax.fori_loop(0, _NCHUNK, chunk, 0)


@functools.cache
def _get_enc_call():
    return pl.kernel(
        _enc_body,
        out_type=jax.ShapeDtypeStruct((_N_PTS, _ENC_DIM), jnp.float32),
        mesh=plsc.VectorSubcoreMesh(core_axis_name="c", subcore_axis_name="s"),
        compiler_params=pltpu.CompilerParams(
            needs_layout_passes=False, use_tc_tiling_on_sc=False),
        scratch_types=[
            pltpu.VMEM((_C,), jnp.float32),
            pltpu.VMEM((_C,), jnp.float32),
            pltpu.VMEM((_C,), jnp.float32),
            pltpu.VMEM((16 * _C,), jnp.int32),
            pltpu.VMEM((16 * _C,), jnp.float32),
            pltpu.VMEM((16 * _C,), jnp.float32),
            pltpu.VMEM((_C, _ENC_DIM), jnp.float32),
            pltpu.SemaphoreType.DMA,
        ],
    )


_BN = 2048


def _mlp_body(enc_ref, w1_ref, w2_ref, o_ref):
    h = jnp.dot(enc_ref[...], w1_ref[...], preferred_element_type=jnp.float32)
    h = jnp.maximum(h, 0.0)
    o = jnp.dot(h, w2_ref[...], preferred_element_type=jnp.float32)
    o_ref[...] = jax.nn.sigmoid(o)


_mlp_call = pl.pallas_call(
    _mlp_body,
    grid=(_N_PTS // _BN,),
    in_specs=[
        pl.BlockSpec((_BN, _ENC_DIM), lambda i: (i, 0)),
        pl.BlockSpec((_ENC_DIM, 64), lambda i: (0, 0)),
        pl.BlockSpec((64, _OUT_DIM), lambda i: (0, 0)),
    ],
    out_specs=pl.BlockSpec((_BN, _OUT_DIM), lambda i: (i, 0)),
    out_shape=jax.ShapeDtypeStruct((_N_PTS, _OUT_DIM), jnp.float32),
)


def kernel(x, tables, W1, W2):
    xs = x[:, 0]
    ys = x[:, 1]
    zs = x[:, 2]
    tab = tables.reshape(_N_LEVELS * _T * _F)
    enc = _get_enc_call()(xs, ys, zs, tab)
    return _mlp_call(enc, W1, W2)
